# Initial kernel scaffold; baseline (speedup 1.0000x reference)
import jax
import jax.numpy as jnp
from jax.experimental import pallas as pl


def kernel(x, edge_index, W1, b1, g1, beta1, W2, b2, g2, beta2):
    raise NotImplementedError("write your pallas kernel here")


# speedup vs baseline: 7.9322x; 7.9322x over previous
import functools

import jax
import jax.numpy as jnp
from jax import lax
from jax.experimental import pallas as pl
from jax.experimental.pallas import tpu as pltpu
from jax.experimental.pallas import tpu_sc as plsc

_N = 10000
_E = 160000
_D = 256
_D2 = 512
_C = 128
_CHUNK = 128
_NCH = _E // _CHUNK
_NS = 16
_ROWS_A = 632
_ROWS_B = _N - 15 * _ROWS_A


def _sc_mesh():
    return plsc.VectorSubcoreMesh(core_axis_name="c", subcore_axis_name="s")


def _for_tile_rows(s, fn):
    @pl.when(s < 15)
    def _():
        fn(pl.multiple_of(s * _ROWS_A, 8), _ROWS_A)

    @pl.when(s == 15)
    def _():
        fn(15 * _ROWS_A, _ROWS_B)


def _deg_call(dst, ones128, zeros):
    @functools.partial(
        pl.kernel,
        mesh=_sc_mesh(),
        out_type=jax.ShapeDtypeStruct((2, _N, _C), jnp.float32),
        scratch_types=[
            pltpu.VMEM((1, _CHUNK), jnp.int32),
            pltpu.VMEM((_CHUNK, _C), jnp.float32),
            pltpu.VMEM_SHARED((_N, _C), jnp.float32),
        ],
    )
    def deg_k(dst_hbm, ones_hbm, zeros_hbm, out_hbm, dstb, ones_v, deg_sh):
        c = lax.axis_index("c")
        s = lax.axis_index("s")
        wid = s * 2 + c

        def init(off, rows):
            pltpu.sync_copy(zeros_hbm.at[pl.ds(off, rows), :],
                            deg_sh.at[pl.ds(off, rows), :])

        _for_tile_rows(s, init)
        pltpu.sync_copy(ones_hbm, ones_v)
        plsc.subcore_barrier()

        def body(j, carry):
            g = j * 32 + wid

            @pl.when(g < _NCH)
            def _():
                pltpu.sync_copy(dst_hbm.at[pl.ds(g * _CHUNK, _CHUNK)],
                                dstb.at[0])
                pltpu.sync_copy(ones_v, deg_sh.at[dstb.at[0]], add=True)

            return carry

        lax.fori_loop(0, (_NCH + 31) // 32, body, 0)
        plsc.subcore_barrier()

        def writeout(off, rows):
            pltpu.sync_copy(deg_sh.at[pl.ds(off, rows), :],
                            out_hbm.at[c, pl.ds(off, rows), :])

        _for_tile_rows(s, writeout)

    return deg_k(dst, ones128, zeros)


def _agg_call(hs_blocked, src, dst, zeros, K):
    @functools.partial(
        pl.kernel,
        mesh=_sc_mesh(),
        out_type=jax.ShapeDtypeStruct((K, _N, _C), jnp.float32),
        scratch_types=[
            pltpu.VMEM((_CHUNK,), jnp.int32),
            pltpu.VMEM((1, _CHUNK), jnp.int32),
            pltpu.VMEM((_CHUNK, _C), jnp.float32),
            pltpu.VMEM_SHARED((_N, _C), jnp.float32),
            pltpu.SemaphoreType.DMA,
        ],
    )
    def agg_k(hs_hbm, src_hbm, dst_hbm, zeros_hbm, out_hbm,
              srcb, dstb, rows, agg_sh, sem):
        c = lax.axis_index("c")
        s = lax.axis_index("s")
        for k in range(K):
            @pl.when(c == (k % 2))
            def _(k=k):
                def init(off, rows_n):
                    pltpu.sync_copy(zeros_hbm.at[pl.ds(off, rows_n), :],
                                    agg_sh.at[pl.ds(off, rows_n), :])

                _for_tile_rows(s, init)
                plsc.subcore_barrier()

                def body(j, carry):
                    g = j * _NS + s

                    @pl.when(g < _NCH)
                    def _():
                        pltpu.sync_copy(src_hbm.at[pl.ds(g * _CHUNK, _CHUNK)],
                                        srcb)
                        pltpu.sync_copy(dst_hbm.at[pl.ds(g * _CHUNK, _CHUNK)],
                                        dstb.at[0])
                        pltpu.async_copy(hs_hbm.at[k].at[srcb], rows,
                                         sem).wait()
                        pltpu.sync_copy(rows, agg_sh.at[dstb.at[0]], add=True)

                    return carry

                lax.fori_loop(0, (_NCH + _NS - 1) // _NS, body, 0)
                plsc.subcore_barrier()

                def writeout(off, rows_n):
                    pltpu.sync_copy(agg_sh.at[pl.ds(off, rows_n), :],
                                    out_hbm.at[k, pl.ds(off, rows_n), :])

                _for_tile_rows(s, writeout)

    return agg_k(hs_blocked, src, dst, zeros)


def _mm_scale_call(xb, w_r, degp, K_in, C_in, KO, R=1000):
    nt = _N // R

    def body(x_ref, w_ref, degp_ref, o_ref):
        acc = jnp.dot(x_ref[0], w_ref[0], preferred_element_type=jnp.float32)
        for k in range(1, K_in):
            acc = acc + jnp.dot(x_ref[k], w_ref[k],
                                preferred_element_type=jnp.float32)
        deg = degp_ref[0, :, 0:1] + degp_ref[1, :, 0:1] + 1.0
        o_ref[0] = acc * lax.rsqrt(deg)

    return pl.pallas_call(
        body,
        grid=(nt, KO),
        in_specs=[
            pl.BlockSpec((K_in, R, C_in), lambda i, j: (0, i, 0)),
            pl.BlockSpec((K_in, C_in, _C), lambda i, j: (0, 0, j)),
            pl.BlockSpec((2, R, _C), lambda i, j: (0, i, 0)),
        ],
        out_specs=pl.BlockSpec((1, R, _C), lambda i, j: (j, i, 0)),
        out_shape=jax.ShapeDtypeStruct((KO, _N, _C), jnp.float32),
    )(xb, w_r, degp)


def _combine_call(agg, hs, degp, b, g, beta, K, blocked_out, R=1000):
    nt = _N // R

    def body(agg_ref, hs_ref, degp_ref, b_ref, g_ref, beta_ref, o_ref):
        deg = degp_ref[0, :, 0:1] + degp_ref[1, :, 0:1] + 1.0
        dis = lax.rsqrt(deg)
        t = (agg_ref[...] + hs_ref[...]) * dis[None] + b_ref[...]
        t = jnp.maximum(t, 0.0)
        mu = jnp.mean(t, axis=(0, 2), keepdims=True)
        d = t - mu
        var = jnp.mean(d * d, axis=(0, 2), keepdims=True)
        t = d * lax.rsqrt(var + 1e-5) * g_ref[...] + beta_ref[...]
        if blocked_out:
            o_ref[...] = t
        else:
            o_ref[...] = jnp.concatenate([t[k] for k in range(K)], axis=-1)

    blk = pl.BlockSpec((K, R, _C), lambda i: (0, i, 0))
    vec = pl.BlockSpec((K, 1, _C), lambda i: (0, 0, 0))
    if blocked_out:
        out_spec = pl.BlockSpec((K, R, _C), lambda i: (0, i, 0))
        out_shape = jax.ShapeDtypeStruct((K, _N, _C), jnp.float32)
    else:
        out_spec = pl.BlockSpec((R, K * _C), lambda i: (i, 0))
        out_shape = jax.ShapeDtypeStruct((_N, K * _C), jnp.float32)
    return pl.pallas_call(
        body,
        grid=(nt,),
        in_specs=[blk, blk, pl.BlockSpec((2, R, _C), lambda i: (0, i, 0)),
                  vec, vec, vec],
        out_specs=out_spec,
        out_shape=out_shape,
    )(agg, hs, degp, b, g, beta)


def kernel(x, edge_index, W1, b1, g1, beta1, W2, b2, g2, beta2):
    src = edge_index[0]
    dst = edge_index[1]
    zeros = jnp.zeros((_N, _C), jnp.float32)
    ones128 = jnp.ones((_CHUNK, _C), jnp.float32)

    degp = _deg_call(dst, ones128, zeros)

    hs1 = _mm_scale_call(x.reshape(1, _N, _D), W1.reshape(1, _D, _D2), degp,
                         K_in=1, C_in=_D, KO=_D2 // _C)
    agg1 = _agg_call(hs1, src, dst, zeros, K=_D2 // _C)
    y1 = _combine_call(agg1, hs1, degp,
                       b1.reshape(_D2 // _C, 1, _C),
                       g1.reshape(_D2 // _C, 1, _C),
                       beta1.reshape(_D2 // _C, 1, _C),
                       K=_D2 // _C, blocked_out=True)

    hs2 = _mm_scale_call(y1, W2.reshape(_D2 // _C, _C, _D), degp,
                         K_in=_D2 // _C, C_in=_C, KO=_D // _C)
    agg2 = _agg_call(hs2, src, dst, zeros, K=_D // _C)
    out = _combine_call(agg2, hs2, degp,
                        b2.reshape(_D // _C, 1, _C),
                        g2.reshape(_D // _C, 1, _C),
                        beta2.reshape(_D // _C, 1, _C),
                        K=_D // _C, blocked_out=False)
    return out


# speedup vs baseline: 11.5336x; 1.4540x over previous
import functools

import jax
import jax.numpy as jnp
from jax import lax
from jax.experimental import pallas as pl
from jax.experimental.pallas import tpu as pltpu
from jax.experimental.pallas import tpu_sc as plsc

_N = 10000
_E = 160000
_D = 256
_D2 = 512
_C = 128
_CHUNK = 128
_NCH = _E // _CHUNK
_NS = 16
_ROWS_A = 632
_ROWS_B = _N - 15 * _ROWS_A


def _sc_mesh():
    return plsc.VectorSubcoreMesh(core_axis_name="c", subcore_axis_name="s")


def _for_tile_rows(s, fn):
    @pl.when(s < 15)
    def _():
        fn(pl.multiple_of(s * _ROWS_A, 8), _ROWS_A)

    @pl.when(s == 15)
    def _():
        fn(15 * _ROWS_A, _ROWS_B)


def _deg_call(dst, ones128, zeros):
    @functools.partial(
        pl.kernel,
        mesh=_sc_mesh(),
        out_type=jax.ShapeDtypeStruct((2, _N, _C), jnp.float32),
        scratch_types=[
            pltpu.VMEM((1, _CHUNK), jnp.int32),
            pltpu.VMEM((_CHUNK, _C), jnp.float32),
            pltpu.VMEM_SHARED((_N, _C), jnp.float32),
        ],
    )
    def deg_k(dst_hbm, ones_hbm, zeros_hbm, out_hbm, dstb, ones_v, deg_sh):
        c = lax.axis_index("c")
        s = lax.axis_index("s")
        wid = s * 2 + c

        def init(off, rows):
            pltpu.sync_copy(zeros_hbm.at[pl.ds(off, rows), :],
                            deg_sh.at[pl.ds(off, rows), :])

        _for_tile_rows(s, init)
        pltpu.sync_copy(ones_hbm, ones_v)
        plsc.subcore_barrier()

        def body(j, carry):
            g = j * 32 + wid

            @pl.when(g < _NCH)
            def _():
                pltpu.sync_copy(dst_hbm.at[pl.ds(g * _CHUNK, _CHUNK)],
                                dstb.at[0])
                pltpu.sync_copy(ones_v, deg_sh.at[dstb.at[0]], add=True)

            return carry

        lax.fori_loop(0, (_NCH + 31) // 32, body, 0)
        plsc.subcore_barrier()

        def writeout(off, rows):
            pltpu.sync_copy(deg_sh.at[pl.ds(off, rows), :],
                            out_hbm.at[c, pl.ds(off, rows), :])

        _for_tile_rows(s, writeout)

    return deg_k(dst, ones128, zeros)


def _agg_call(hs_blocked, src, dst, zeros, K):
    @functools.partial(
        pl.kernel,
        mesh=_sc_mesh(),
        out_type=jax.ShapeDtypeStruct((K, _N, _C), jnp.float32),
        scratch_types=[
            pltpu.VMEM((2, _CHUNK), jnp.int32),
            pltpu.VMEM((2, _CHUNK), jnp.int32),
            pltpu.VMEM((2, _CHUNK, _C), jnp.float32),
            pltpu.VMEM_SHARED((_N, _C), jnp.float32),
            pltpu.SemaphoreType.DMA,
            pltpu.SemaphoreType.DMA,
        ],
    )
    def agg_k(hs_hbm, src_hbm, dst_hbm, zeros_hbm, out_hbm,
              srcb, dstb, rows, agg_sh, sem0, sem1):
        c = lax.axis_index("c")
        s = lax.axis_index("s")
        nj = (_NCH + _NS - 1) // _NS
        for k in range(K):
            @pl.when(c == (k % 2))
            def _(k=k):
                sems = (sem0, sem1)

                def init(off, rows_n):
                    pltpu.sync_copy(zeros_hbm.at[pl.ds(off, rows_n), :],
                                    agg_sh.at[pl.ds(off, rows_n), :])

                _for_tile_rows(s, init)
                plsc.subcore_barrier()

                def load_and_gather(j, b):
                    g = j * _NS + s

                    @pl.when(g < _NCH)
                    def _():
                        pltpu.sync_copy(src_hbm.at[pl.ds(g * _CHUNK, _CHUNK)],
                                        srcb.at[b])
                        pltpu.sync_copy(dst_hbm.at[pl.ds(g * _CHUNK, _CHUNK)],
                                        dstb.at[b])
                        pltpu.async_copy(hs_hbm.at[k].at[srcb.at[b]],
                                         rows.at[b], sems[b])

                def drain_and_scatter(j, b):
                    g = j * _NS + s

                    @pl.when(g < _NCH)
                    def _():
                        pltpu.make_async_copy(hs_hbm.at[k].at[srcb.at[b]],
                                              rows.at[b], sems[b]).wait()
                        pltpu.sync_copy(rows.at[b], agg_sh.at[dstb.at[b]],
                                        add=True)

                load_and_gather(0, 0)

                def body(j, carry):
                    load_and_gather(2 * j + 1, 1)
                    drain_and_scatter(2 * j, 0)
                    load_and_gather(2 * j + 2, 0)
                    drain_and_scatter(2 * j + 1, 1)
                    return carry

                lax.fori_loop(0, nj // 2, body, 0)
                if nj % 2:
                    drain_and_scatter(nj - 1, 0)
                plsc.subcore_barrier()

                def writeout(off, rows_n):
                    pltpu.sync_copy(agg_sh.at[pl.ds(off, rows_n), :],
                                    out_hbm.at[k, pl.ds(off, rows_n), :])

                _for_tile_rows(s, writeout)

    return agg_k(hs_blocked, src, dst, zeros)


def _mm_scale_call(xb, w_r, degp, K_in, C_in, KO, R=1000):
    nt = _N // R

    def body(x_ref, w_ref, degp_ref, o_ref):
        acc = jnp.dot(x_ref[0], w_ref[0], preferred_element_type=jnp.float32)
        for k in range(1, K_in):
            acc = acc + jnp.dot(x_ref[k], w_ref[k],
                                preferred_element_type=jnp.float32)
        deg = degp_ref[0, :, 0:1] + degp_ref[1, :, 0:1] + 1.0
        o_ref[0] = acc * lax.rsqrt(deg)

    return pl.pallas_call(
        body,
        grid=(nt, KO),
        in_specs=[
            pl.BlockSpec((K_in, R, C_in), lambda i, j: (0, i, 0)),
            pl.BlockSpec((K_in, C_in, _C), lambda i, j: (0, 0, j)),
            pl.BlockSpec((2, R, _C), lambda i, j: (0, i, 0)),
        ],
        out_specs=pl.BlockSpec((1, R, _C), lambda i, j: (j, i, 0)),
        out_shape=jax.ShapeDtypeStruct((KO, _N, _C), jnp.float32),
    )(xb, w_r, degp)


def _combine_call(agg, hs, degp, b, g, beta, K, blocked_out, R=1000):
    nt = _N // R

    def body(agg_ref, hs_ref, degp_ref, b_ref, g_ref, beta_ref, o_ref):
        deg = degp_ref[0, :, 0:1] + degp_ref[1, :, 0:1] + 1.0
        dis = lax.rsqrt(deg)
        t = (agg_ref[...] + hs_ref[...]) * dis[None] + b_ref[...]
        t = jnp.maximum(t, 0.0)
        mu = jnp.mean(t, axis=(0, 2), keepdims=True)
        d = t - mu
        var = jnp.mean(d * d, axis=(0, 2), keepdims=True)
        t = d * lax.rsqrt(var + 1e-5) * g_ref[...] + beta_ref[...]
        if blocked_out:
            o_ref[...] = t
        else:
            o_ref[...] = jnp.concatenate([t[k] for k in range(K)], axis=-1)

    blk = pl.BlockSpec((K, R, _C), lambda i: (0, i, 0))
    vec = pl.BlockSpec((K, 1, _C), lambda i: (0, 0, 0))
    if blocked_out:
        out_spec = pl.BlockSpec((K, R, _C), lambda i: (0, i, 0))
        out_shape = jax.ShapeDtypeStruct((K, _N, _C), jnp.float32)
    else:
        out_spec = pl.BlockSpec((R, K * _C), lambda i: (i, 0))
        out_shape = jax.ShapeDtypeStruct((_N, K * _C), jnp.float32)
    return pl.pallas_call(
        body,
        grid=(nt,),
        in_specs=[blk, blk, pl.BlockSpec((2, R, _C), lambda i: (0, i, 0)),
                  vec, vec, vec],
        out_specs=out_spec,
        out_shape=out_shape,
    )(agg, hs, degp, b, g, beta)


def kernel(x, edge_index, W1, b1, g1, beta1, W2, b2, g2, beta2):
    src = edge_index[0]
    dst = edge_index[1]
    zeros = jnp.zeros((_N, _C), jnp.float32)
    ones128 = jnp.ones((_CHUNK, _C), jnp.float32)

    degp = _deg_call(dst, ones128, zeros)

    hs1 = _mm_scale_call(x.reshape(1, _N, _D), W1.reshape(1, _D, _D2), degp,
                         K_in=1, C_in=_D, KO=_D2 // _C)
    agg1 = _agg_call(hs1, src, dst, zeros, K=_D2 // _C)
    y1 = _combine_call(agg1, hs1, degp,
                       b1.reshape(_D2 // _C, 1, _C),
                       g1.reshape(_D2 // _C, 1, _C),
                       beta1.reshape(_D2 // _C, 1, _C),
                       K=_D2 // _C, blocked_out=True)

    hs2 = _mm_scale_call(y1, W2.reshape(_D2 // _C, _C, _D), degp,
                         K_in=_D2 // _C, C_in=_C, KO=_D // _C)
    agg2 = _agg_call(hs2, src, dst, zeros, K=_D // _C)
    out = _combine_call(agg2, hs2, degp,
                        b2.reshape(_D // _C, 1, _C),
                        g2.reshape(_D // _C, 1, _C),
                        beta2.reshape(_D // _C, 1, _C),
                        K=_D // _C, blocked_out=False)
    return out
